# Initial kernel scaffold; baseline (speedup 1.0000x reference)
import jax
import jax.numpy as jnp
from jax.experimental import pallas as pl


def kernel(x1, x2, edge_index, Wl1_w2s, bl1_w2s, Wr1_w2s, Wl1_s2w, bl1_s2w, Wr1_s2w, Wl2_w2s, bl2_w2s, Wr2_w2s, Wl2_s2w, bl2_s2w, Wr2_s2w, Wl3_w2s, bl3_w2s, Wr3_w2s, Wl3_s2w, bl3_s2w, Wr3_s2w, Wfc, bfc):
    raise NotImplementedError("write your pallas kernel here")


# speedup vs baseline: 30.2956x; 30.2956x over previous
import functools

import jax
import jax.numpy as jnp
from jax import lax
from jax.experimental import pallas as pl
from jax.experimental.pallas import tpu as pltpu
from jax.experimental.pallas import tpu_sc as plsc

N1 = 50000
N2 = 50000
E = 1600000
H = 16

NCORES = 2
NSUB = 16
NW = NCORES * NSUB
CH = 128
KC = 8
EW = 50176
NSUP = EW // (CH * KC)
EP = EW * NW
NP = 50176
ROWS_T = NP // NSUB


def _make_sweep(two_dir, w, interpret=False):
    mesh = plsc.VectorSubcoreMesh(core_axis_name="c", subcore_axis_name="s", num_cores=NCORES, num_subcores=NSUB)
    out_type = [jax.ShapeDtypeStruct((NCORES, NP, w), jnp.float32)
                for _ in range(2 if two_dir else 1)]
    scratch = [
        pltpu.VMEM((KC, CH), jnp.int32),
        pltpu.VMEM((KC, CH), jnp.int32),
        pltpu.VMEM((KC, CH, w), jnp.float32),
        pltpu.VMEM_SHARED((NP, w), jnp.float32),
        pltpu.SemaphoreType.DMA,
    ]
    if two_dir:
        scratch += [
            pltpu.VMEM((KC, CH, w), jnp.float32),
            pltpu.VMEM_SHARED((NP, w), jnp.float32),
            pltpu.SemaphoreType.DMA,
        ]

    def body(*refs):
        if two_dir:
            (srcH, dstH, zH, tabdH, tabsH, out1H, out2H,
             sidx, didx, g1, acc1, sem1, g2, acc2, sem2) = refs
        else:
            (srcH, dstH, zH, tabdH, out1H,
             sidx, didx, g1, acc1, sem1) = refs
            tabsH = out2H = g2 = acc2 = sem2 = None
        cid = lax.axis_index("c")
        sid = lax.axis_index("s")
        wid = sid * NCORES + cid
        r0 = sid * ROWS_T
        pltpu.sync_copy(zH.at[pl.ds(r0, ROWS_T), :], acc1.at[pl.ds(r0, ROWS_T), :])
        if two_dir:
            pltpu.sync_copy(zH.at[pl.ds(r0, ROWS_T), :], acc2.at[pl.ds(r0, ROWS_T), :])
        plsc.subcore_barrier()

        row0 = wid * (EW // CH)

        def sup(i, carry):
            rbase = row0 + i * KC
            pltpu.sync_copy(srcH.at[pl.ds(rbase, KC), :], sidx)
            pltpu.sync_copy(dstH.at[pl.ds(rbase, KC), :], didx)
            d1, d2 = [], []
            for k in range(KC):
                d1.append(pltpu.async_copy(tabdH.at[didx.at[k]], g1.at[k], sem1))
            if two_dir:
                for k in range(KC):
                    d2.append(pltpu.async_copy(tabsH.at[sidx.at[k]], g2.at[k], sem2))
            for k in range(KC):
                d1[k].wait()
            if two_dir:
                for k in range(KC):
                    d2[k].wait()
            for k in range(KC):
                pltpu.sync_copy(g1.at[k], acc1.at[sidx.at[k]], add=True)
                if two_dir:
                    pltpu.sync_copy(g2.at[k], acc2.at[didx.at[k]], add=True)
            return carry

        lax.fori_loop(0, NSUP, sup, 0)
        plsc.subcore_barrier()
        pltpu.sync_copy(acc1.at[pl.ds(r0, ROWS_T), :],
                        out1H.at[cid, pl.ds(r0, ROWS_T), :])
        if two_dir:
            pltpu.sync_copy(acc2.at[pl.ds(r0, ROWS_T), :],
                            out2H.at[cid, pl.ds(r0, ROWS_T), :])

    return pl.kernel(body, out_type=tuple(out_type), mesh=mesh,
                     scratch_types=scratch, interpret=interpret,
                     compiler_params=pltpu.CompilerParams(
                         use_tc_tiling_on_sc=False))


def _make_sweep_dir_per_core(w, interpret=False):
    mesh = plsc.VectorSubcoreMesh(core_axis_name="c", subcore_axis_name="s", num_cores=NCORES, num_subcores=NSUB)
    out_type = (jax.ShapeDtypeStruct((NP, w), jnp.float32),
                jax.ShapeDtypeStruct((NP, w), jnp.float32))
    scratch = [
        pltpu.VMEM((KC, CH), jnp.int32),
        pltpu.VMEM((KC, CH), jnp.int32),
        pltpu.VMEM((KC, CH, w), jnp.float32),
        pltpu.VMEM_SHARED((NP, w), jnp.float32),
        pltpu.SemaphoreType.DMA,
    ]
    rows_per_tile = EP // CH // NSUB
    nsup = rows_per_tile // KC

    def body(srcH, dstH, zH, tabdH, tabsH, out1H, out2H,
             sidx, didx, g1, acc, sem1):
        cid = lax.axis_index("c")
        sid = lax.axis_index("s")
        r0 = sid * ROWS_T
        pltpu.sync_copy(zH.at[pl.ds(r0, ROWS_T), :], acc.at[pl.ds(r0, ROWS_T), :])
        plsc.subcore_barrier()
        row0 = sid * rows_per_tile

        def sup(i, carry):
            rbase = row0 + i * KC
            pltpu.sync_copy(srcH.at[pl.ds(rbase, KC), :], sidx)
            pltpu.sync_copy(dstH.at[pl.ds(rbase, KC), :], didx)

            @pl.when(cid == 0)
            def _():
                ds = [pltpu.async_copy(tabdH.at[didx.at[k]], g1.at[k], sem1)
                      for k in range(KC)]
                for k in range(KC):
                    ds[k].wait()
                for k in range(KC):
                    pltpu.sync_copy(g1.at[k], acc.at[sidx.at[k]], add=True)

            @pl.when(cid == 1)
            def _():
                ds = [pltpu.async_copy(tabsH.at[sidx.at[k]], g1.at[k], sem1)
                      for k in range(KC)]
                for k in range(KC):
                    ds[k].wait()
                for k in range(KC):
                    pltpu.sync_copy(g1.at[k], acc.at[didx.at[k]], add=True)

            return carry

        lax.fori_loop(0, nsup, sup, 0)
        plsc.subcore_barrier()

        @pl.when(cid == 0)
        def _():
            pltpu.sync_copy(acc.at[pl.ds(r0, ROWS_T), :],
                            out1H.at[pl.ds(r0, ROWS_T), :])

        @pl.when(cid == 1)
        def _():
            pltpu.sync_copy(acc.at[pl.ds(r0, ROWS_T), :],
                            out2H.at[pl.ds(r0, ROWS_T), :])

    return pl.kernel(body, out_type=out_type, mesh=mesh,
                     scratch_types=scratch, interpret=interpret,
                     compiler_params=pltpu.CompilerParams(
                         use_tc_tiling_on_sc=False))


_sweeps = None


def _get_sweeps():
    global _sweeps
    if _sweeps is None:
        _sweeps = (_make_sweep_dir_per_core(H),
                   _make_sweep_dir_per_core(H),
                   _make_sweep(False, H))
    return _sweeps


BLK = 1024
GRID = NP // BLK


def _full(shape):
    return pl.BlockSpec(shape, lambda i: tuple(0 for _ in shape))


def _d1_body(acc_ref, x_ref, wl_ref, wr_ref, b_ref, h_ref, rd_ref):
    acc = acc_ref[...]
    s = acc[:, 0:1]
    dg = acc[:, 1:2]
    rd = 1.0 / jnp.maximum(dg, 1.0)
    h = (s * rd) * wl_ref[...] + x_ref[...] * wr_ref[...] + b_ref[...]
    h_ref[...] = jnp.maximum(h, 0.0)
    rd_ref[...] = rd


def _dense1(acc, x, wl, wr, b):
    return pl.pallas_call(
        _d1_body,
        grid=(GRID,),
        in_specs=[
            pl.BlockSpec((BLK, H), lambda i: (i, 0)),
            pl.BlockSpec((BLK, 1), lambda i: (i, 0)),
            _full((1, H)), _full((1, H)), _full((1, H)),
        ],
        out_specs=[
            pl.BlockSpec((BLK, H), lambda i: (i, 0)),
            pl.BlockSpec((BLK, 1), lambda i: (i, 0)),
        ],
        out_shape=[
            jax.ShapeDtypeStruct((NP, H), jnp.float32),
            jax.ShapeDtypeStruct((NP, 1), jnp.float32),
        ],
    )(acc, x, wl, wr, b)


def _d2_body(acc_ref, rd_ref, hp_ref, wlT_ref, wrT_ref, b_ref, h_ref):
    m = acc_ref[...] * rd_ref[...]
    h = (jnp.dot(m, wlT_ref[...], preferred_element_type=jnp.float32)
         + jnp.dot(hp_ref[...], wrT_ref[...], preferred_element_type=jnp.float32)
         + b_ref[...])
    h_ref[...] = jnp.maximum(h, 0.0)


def _dense2(acc, rd, hp, wlT, wrT, b):
    return pl.pallas_call(
        _d2_body,
        grid=(GRID,),
        in_specs=[
            pl.BlockSpec((BLK, H), lambda i: (i, 0)),
            pl.BlockSpec((BLK, 1), lambda i: (i, 0)),
            pl.BlockSpec((BLK, H), lambda i: (i, 0)),
            _full((H, H)), _full((H, H)), _full((1, H)),
        ],
        out_specs=pl.BlockSpec((BLK, H), lambda i: (i, 0)),
        out_shape=jax.ShapeDtypeStruct((NP, H), jnp.float32),
    )(acc, rd, hp, wlT, wrT, b)


def _d3_body(acc_ref, rd_ref, hp_ref, wlT_ref, wrT_ref, b_ref,
             wfcT_ref, bfc_ref, o_ref):
    s = acc_ref[0] + acc_ref[1]
    m = s * rd_ref[...]
    g = (jnp.dot(m, wlT_ref[...], preferred_element_type=jnp.float32)
         + jnp.dot(hp_ref[...], wrT_ref[...], preferred_element_type=jnp.float32)
         + b_ref[...])
    g = jnp.maximum(g, 0.0)
    z = jnp.dot(g, wfcT_ref[...], preferred_element_type=jnp.float32) + bfc_ref[...]
    o_ref[...] = 1.0 / (1.0 + jnp.exp(-z))


def _dense3(acc, rd, hp, wlT, wrT, b, wfcT, bfc):
    return pl.pallas_call(
        _d3_body,
        grid=(GRID,),
        in_specs=[
            pl.BlockSpec((NCORES, BLK, H), lambda i: (0, i, 0)),
            pl.BlockSpec((BLK, 1), lambda i: (i, 0)),
            pl.BlockSpec((BLK, H), lambda i: (i, 0)),
            _full((H, H)), _full((H, H)), _full((1, H)),
            _full((H, 1)), _full((1, 1)),
        ],
        out_specs=pl.BlockSpec((BLK, 1), lambda i: (i, 0)),
        out_shape=jax.ShapeDtypeStruct((NP, 1), jnp.float32),
    )(acc, rd, hp, wlT, wrT, b, wfcT, bfc)


def kernel(x1, x2, edge_index,
           Wl1_w2s, bl1_w2s, Wr1_w2s, Wl1_s2w, bl1_s2w, Wr1_s2w,
           Wl2_w2s, bl2_w2s, Wr2_w2s, Wl2_s2w, bl2_s2w, Wr2_s2w,
           Wl3_w2s, bl3_w2s, Wr3_w2s, Wl3_s2w, bl3_s2w, Wr3_s2w,
           Wfc, bfc):
    sweep2, sweep16_two, sweep16_one = _get_sweeps()

    src = edge_index[0].astype(jnp.int32)
    dst = edge_index[1].astype(jnp.int32)
    src2d = jnp.concatenate(
        [src, jnp.full((EP - E,), N1, jnp.int32)]).reshape(EP // CH, CH)
    dst2d = jnp.concatenate(
        [dst, jnp.full((EP - E,), N2, jnp.int32)]).reshape(EP // CH, CH)

    zeros16 = jnp.zeros((NP, H), jnp.float32)

    ones1 = jnp.ones((N1, 1), jnp.float32)
    t1 = jnp.concatenate([jnp.concatenate(
        [x1, ones1, jnp.zeros((N1, H - 2), jnp.float32)], axis=1),
        jnp.zeros((NP - N1, H), jnp.float32)], axis=0)
    t2 = jnp.concatenate([jnp.concatenate(
        [x2, ones1, jnp.zeros((N2, H - 2), jnp.float32)], axis=1),
        jnp.zeros((NP - N2, H), jnp.float32)], axis=0)
    x1p = jnp.concatenate([x1, jnp.zeros((NP - N1, 1), jnp.float32)], axis=0)
    x2p = jnp.concatenate([x2, jnp.zeros((NP - N2, 1), jnp.float32)], axis=0)

    accA1, accA2 = sweep2(src2d, dst2d, zeros16, t2, t1)

    h1, rd1 = _dense1(accA1, x1p, Wl1_w2s.T.reshape(1, H),
                      Wr1_w2s.T.reshape(1, H), bl1_w2s.reshape(1, H))
    h2, rd2 = _dense1(accA2, x2p, Wl1_s2w.T.reshape(1, H),
                      Wr1_s2w.T.reshape(1, H), bl1_s2w.reshape(1, H))

    accB1, accB2 = sweep16_two(src2d, dst2d, zeros16, h2, h1)
    h1b = _dense2(accB1, rd1, h1, Wl2_w2s.T, Wr2_w2s.T, bl2_w2s.reshape(1, H))
    h2b = _dense2(accB2, rd2, h2, Wl2_s2w.T, Wr2_s2w.T, bl2_s2w.reshape(1, H))

    (accC1,) = sweep16_one(src2d, dst2d, zeros16, h2b)
    o = _dense3(accC1, rd1, h1b, Wl3_w2s.T, Wr3_w2s.T, bl3_w2s.reshape(1, H),
                Wfc.T, bfc.reshape(1, 1))
    return o[:N1, 0]


# speedup vs baseline: 57.1765x; 1.8873x over previous
import jax
import jax.numpy as jnp
from jax import lax
from jax.experimental import pallas as pl
from jax.experimental.pallas import tpu as pltpu
from jax.experimental.pallas import tpu_sc as plsc

N1 = 50000
N2 = 50000
E = 1600000
H = 16

NCORES = 2
NSUB = 16
NW = NCORES * NSUB
CH = 128
KC = 8
EW = 50176
NSUP = EW // (CH * KC)
EP = EW * NW
NP = 50176
ROWS_T = NP // NSUB


def _make_sweep(two_dir, w, interpret=False):
    mesh = plsc.VectorSubcoreMesh(core_axis_name="c", subcore_axis_name="s", num_cores=NCORES, num_subcores=NSUB)
    out_type = [jax.ShapeDtypeStruct((NCORES, NP, w), jnp.float32)
                for _ in range(2 if two_dir else 1)]
    scratch = [
        pltpu.VMEM((KC, CH), jnp.int32),
        pltpu.VMEM((KC, CH), jnp.int32),
        pltpu.VMEM((KC, CH, w), jnp.float32),
        pltpu.VMEM_SHARED((NP, w), jnp.float32),
        pltpu.SemaphoreType.DMA,
    ]
    if two_dir:
        scratch += [
            pltpu.VMEM((KC, CH, w), jnp.float32),
            pltpu.VMEM_SHARED((NP, w), jnp.float32),
            pltpu.SemaphoreType.DMA,
        ]

    def body(*refs):
        if two_dir:
            (srcH, dstH, zH, tabdH, tabsH, out1H, out2H,
             sidx, didx, g1, acc1, sem1, g2, acc2, sem2) = refs
        else:
            (srcH, dstH, zH, tabdH, out1H,
             sidx, didx, g1, acc1, sem1) = refs
            tabsH = out2H = g2 = acc2 = sem2 = None
        cid = lax.axis_index("c")
        sid = lax.axis_index("s")
        wid = sid * NCORES + cid
        r0 = sid * ROWS_T
        pltpu.sync_copy(zH.at[pl.ds(r0, ROWS_T), :], acc1.at[pl.ds(r0, ROWS_T), :])
        if two_dir:
            pltpu.sync_copy(zH.at[pl.ds(r0, ROWS_T), :], acc2.at[pl.ds(r0, ROWS_T), :])
        plsc.subcore_barrier()

        row0 = wid * (EW // CH)

        def sup(i, carry):
            rbase = row0 + i * KC
            pltpu.sync_copy(srcH.at[pl.ds(rbase, KC), :], sidx)
            pltpu.sync_copy(dstH.at[pl.ds(rbase, KC), :], didx)
            d1, d2 = [], []
            for k in range(KC):
                d1.append(pltpu.async_copy(tabdH.at[didx.at[k]], g1.at[k], sem1))
            if two_dir:
                for k in range(KC):
                    d2.append(pltpu.async_copy(tabsH.at[sidx.at[k]], g2.at[k], sem2))
            for k in range(KC):
                d1[k].wait()
            if two_dir:
                for k in range(KC):
                    d2[k].wait()
            for k in range(KC):
                pltpu.sync_copy(g1.at[k], acc1.at[sidx.at[k]], add=True)
                if two_dir:
                    pltpu.sync_copy(g2.at[k], acc2.at[didx.at[k]], add=True)
            return carry

        lax.fori_loop(0, NSUP, sup, 0)
        plsc.subcore_barrier()
        pltpu.sync_copy(acc1.at[pl.ds(r0, ROWS_T), :],
                        out1H.at[cid, pl.ds(r0, ROWS_T), :])
        if two_dir:
            pltpu.sync_copy(acc2.at[pl.ds(r0, ROWS_T), :],
                            out2H.at[cid, pl.ds(r0, ROWS_T), :])

    return pl.kernel(body, out_type=tuple(out_type), mesh=mesh,
                     scratch_types=scratch, interpret=interpret,
                     compiler_params=pltpu.CompilerParams(
                         use_tc_tiling_on_sc=False))


def _make_sweep_dir_per_core(w, interpret=False):
    mesh = plsc.VectorSubcoreMesh(core_axis_name="c", subcore_axis_name="s",
                                  num_cores=NCORES, num_subcores=NSUB)
    out_type = (jax.ShapeDtypeStruct((NP, w), jnp.float32),
                jax.ShapeDtypeStruct((NP, w), jnp.float32))
    scratch = [
        pltpu.VMEM((2, KC, CH), jnp.int32),
        pltpu.VMEM((2, KC, CH), jnp.int32),
        pltpu.VMEM((2, KC, CH, w), jnp.float32),
        pltpu.VMEM_SHARED((NP, w), jnp.float32),
        pltpu.SemaphoreType.DMA,
        pltpu.SemaphoreType.DMA,
        pltpu.SemaphoreType.DMA,
        pltpu.SemaphoreType.DMA,
    ]
    rows_per_tile = EP // CH // NSUB
    nsup = rows_per_tile // KC
    npair = nsup // 2

    def body(srcH, dstH, zH, tabdH, tabsH, out1H, out2H,
             sidx, didx, g, acc, gs0, gs1, ss0, ss1):
        cid = lax.axis_index("c")
        sid = lax.axis_index("s")
        r0 = sid * ROWS_T
        pltpu.sync_copy(zH.at[pl.ds(r0, ROWS_T), :], acc.at[pl.ds(r0, ROWS_T), :])
        plsc.subcore_barrier()
        row0 = sid * rows_per_tile
        gsem = (gs0, gs1)
        ssem = (ss0, ss1)

        def pipe(gidx, scidx, tabH):
            def fire(b, rbase):
                pltpu.sync_copy(srcH.at[pl.ds(rbase, KC), :], sidx.at[b])
                pltpu.sync_copy(dstH.at[pl.ds(rbase, KC), :], didx.at[b])
                for k in range(KC):
                    pltpu.async_copy(tabH.at[gidx.at[b, k]], g.at[b, k], gsem[b])

            def drain_gather(b):
                for k in range(KC):
                    pltpu.make_async_copy(tabH.at[gidx.at[b, k]], g.at[b, k],
                                          gsem[b]).wait()

            def fire_scatter(b):
                for k in range(KC):
                    pltpu.async_copy(g.at[b, k], acc.at[scidx.at[b, k]],
                                     ssem[b], add=True)

            def drain_scatter(b):
                for k in range(KC):
                    pltpu.make_async_copy(g.at[b, k], acc.at[scidx.at[b, k]],
                                          ssem[b]).wait()

            def pair(i, carry):
                rbase = row0 + i * 2 * KC

                @pl.when(i > 0)
                def _():
                    drain_scatter(0)
                fire(0, rbase)

                @pl.when(i > 0)
                def _():
                    drain_scatter(1)
                fire(1, rbase + KC)

                drain_gather(0)
                fire_scatter(0)
                drain_gather(1)
                fire_scatter(1)
                return carry

            lax.fori_loop(0, npair, pair, 0)
            drain_scatter(0)
            drain_scatter(1)

        @pl.when(cid == 0)
        def _():
            pipe(didx, sidx, tabdH)

        @pl.when(cid == 1)
        def _():
            pipe(sidx, didx, tabsH)

        plsc.subcore_barrier()

        @pl.when(cid == 0)
        def _():
            pltpu.sync_copy(acc.at[pl.ds(r0, ROWS_T), :],
                            out1H.at[pl.ds(r0, ROWS_T), :])

        @pl.when(cid == 1)
        def _():
            pltpu.sync_copy(acc.at[pl.ds(r0, ROWS_T), :],
                            out2H.at[pl.ds(r0, ROWS_T), :])

    return pl.kernel(body, out_type=out_type, mesh=mesh,
                     scratch_types=scratch, interpret=interpret,
                     compiler_params=pltpu.CompilerParams(
                         use_tc_tiling_on_sc=False))


def _make_sweep_one_dir(w, interpret=False):
    mesh = plsc.VectorSubcoreMesh(core_axis_name="c", subcore_axis_name="s",
                                  num_cores=NCORES, num_subcores=NSUB)
    out_type = (jax.ShapeDtypeStruct((NCORES, NP, w), jnp.float32),)
    scratch = [
        pltpu.VMEM((2, KC, CH), jnp.int32),
        pltpu.VMEM((2, KC, CH), jnp.int32),
        pltpu.VMEM((2, KC, CH, w), jnp.float32),
        pltpu.VMEM_SHARED((NP, w), jnp.float32),
        pltpu.SemaphoreType.DMA,
        pltpu.SemaphoreType.DMA,
        pltpu.SemaphoreType.DMA,
        pltpu.SemaphoreType.DMA,
    ]
    rows_per_tile = EW // CH
    npair = rows_per_tile // KC // 2
    tail = rows_per_tile - npair * 2 * KC

    def body(srcH, dstH, zH, tabdH, out1H,
             sidx, didx, g, acc, gs0, gs1, ss0, ss1):
        cid = lax.axis_index("c")
        sid = lax.axis_index("s")
        wid = sid * NCORES + cid
        r0 = sid * ROWS_T
        pltpu.sync_copy(zH.at[pl.ds(r0, ROWS_T), :], acc.at[pl.ds(r0, ROWS_T), :])
        plsc.subcore_barrier()
        row0 = wid * rows_per_tile
        gsem = (gs0, gs1)
        ssem = (ss0, ss1)

        def fire(b, rbase):
            pltpu.sync_copy(srcH.at[pl.ds(rbase, KC), :], sidx.at[b])
            pltpu.sync_copy(dstH.at[pl.ds(rbase, KC), :], didx.at[b])
            for k in range(KC):
                pltpu.async_copy(tabdH.at[didx.at[b, k]], g.at[b, k], gsem[b])

        def drain_gather(b):
            for k in range(KC):
                pltpu.make_async_copy(tabdH.at[didx.at[b, k]], g.at[b, k],
                                      gsem[b]).wait()

        def fire_scatter(b):
            for k in range(KC):
                pltpu.async_copy(g.at[b, k], acc.at[sidx.at[b, k]],
                                 ssem[b], add=True)

        def drain_scatter(b):
            for k in range(KC):
                pltpu.make_async_copy(g.at[b, k], acc.at[sidx.at[b, k]],
                                      ssem[b]).wait()

        def pair(i, carry):
            rbase = row0 + i * 2 * KC

            @pl.when(i > 0)
            def _():
                drain_scatter(0)
            fire(0, rbase)

            @pl.when(i > 0)
            def _():
                drain_scatter(1)
            fire(1, rbase + KC)

            drain_gather(0)
            fire_scatter(0)
            drain_gather(1)
            fire_scatter(1)
            return carry

        lax.fori_loop(0, npair, pair, 0)
        drain_scatter(0)
        drain_scatter(1)
        if tail:
            assert tail == KC
            fire(0, row0 + npair * 2 * KC)
            drain_gather(0)
            fire_scatter(0)
            drain_scatter(0)
        plsc.subcore_barrier()
        pltpu.sync_copy(acc.at[pl.ds(r0, ROWS_T), :],
                        out1H.at[cid, pl.ds(r0, ROWS_T), :])

    return pl.kernel(body, out_type=out_type, mesh=mesh,
                     scratch_types=scratch, interpret=interpret,
                     compiler_params=pltpu.CompilerParams(
                         use_tc_tiling_on_sc=False))


_sweeps = None


def _get_sweeps():
    global _sweeps
    if _sweeps is None:
        _sweeps = (_make_sweep_dir_per_core(H),
                   _make_sweep_dir_per_core(H),
                   _make_sweep_one_dir(H))
    return _sweeps


M8 = NP // 8


def _d1_body(a1_ref, a2_ref, xb1_ref, xb2_ref, s0_ref, s1_ref,
             wl1_ref, wr1_ref, b1_ref, wl2_ref, wr2_ref, b2_ref,
             h1_ref, rd1_ref, h2_ref, rd2_ref):
    s0 = s0_ref[...]
    s1 = s1_ref[...]

    def side(a_ref, xb_ref, wl_ref, wr_ref, b_ref, h_ref, rd_ref):
        a = a_ref[...]
        ssum = jnp.dot(a, s0, preferred_element_type=jnp.float32)
        deg = jnp.dot(a, s1, preferred_element_type=jnp.float32)
        rd = 1.0 / jnp.maximum(deg, 1.0)
        h = (ssum * rd) * wl_ref[...] + xb_ref[...] * wr_ref[...] + b_ref[...]
        h_ref[...] = jnp.maximum(h, 0.0)
        rd_ref[...] = rd

    side(a1_ref, xb1_ref, wl1_ref, wr1_ref, b1_ref, h1_ref, rd1_ref)
    side(a2_ref, xb2_ref, wl2_ref, wr2_ref, b2_ref, h2_ref, rd2_ref)


def _dense1(a1, a2, xb1, xb2, s0, s1, wl1, wr1, b1, wl2, wr2, b2):
    return pl.pallas_call(
        _d1_body,
        out_shape=[
            jax.ShapeDtypeStruct((M8, 128), jnp.float32),
            jax.ShapeDtypeStruct((M8, 128), jnp.float32),
            jax.ShapeDtypeStruct((M8, 128), jnp.float32),
            jax.ShapeDtypeStruct((M8, 128), jnp.float32),
        ],
    )(a1, a2, xb1, xb2, s0, s1, wl1, wr1, b1, wl2, wr2, b2)


def _d2_body(a1_ref, a2_ref, rd1_ref, rd2_ref, h1_ref, h2_ref,
             wl1_ref, wr1_ref, b1_ref, wl2_ref, wr2_ref, b2_ref,
             o1_ref, o2_ref):
    m1 = a1_ref[...] * rd1_ref[...]
    o1 = (jnp.dot(m1, wl1_ref[...], preferred_element_type=jnp.float32)
          + jnp.dot(h1_ref[...], wr1_ref[...], preferred_element_type=jnp.float32)
          + b1_ref[...])
    o1_ref[...] = jnp.maximum(o1, 0.0)
    m2 = a2_ref[...] * rd2_ref[...]
    o2 = (jnp.dot(m2, wl2_ref[...], preferred_element_type=jnp.float32)
          + jnp.dot(h2_ref[...], wr2_ref[...], preferred_element_type=jnp.float32)
          + b2_ref[...])
    o2_ref[...] = jnp.maximum(o2, 0.0)


def _dense2(a1, a2, rd1, rd2, h1, h2, wl1, wr1, b1, wl2, wr2, b2):
    return pl.pallas_call(
        _d2_body,
        out_shape=[
            jax.ShapeDtypeStruct((M8, 128), jnp.float32),
            jax.ShapeDtypeStruct((M8, 128), jnp.float32),
        ],
    )(a1, a2, rd1, rd2, h1, h2, wl1, wr1, b1, wl2, wr2, b2)


def _d3_body(acc_ref, rd_ref, hp_ref, wl_ref, wr_ref, b_ref,
             wfc_ref, bfc_ref, o_ref):
    m = (acc_ref[0] + acc_ref[1]) * rd_ref[...]
    g = (jnp.dot(m, wl_ref[...], preferred_element_type=jnp.float32)
         + jnp.dot(hp_ref[...], wr_ref[...], preferred_element_type=jnp.float32)
         + b_ref[...])
    g = jnp.maximum(g, 0.0)
    z = jnp.dot(g, wfc_ref[...], preferred_element_type=jnp.float32) + bfc_ref[...]
    o_ref[...] = 1.0 / (1.0 + jnp.exp(-z))


def _dense3(acc, rd, hp, wl, wr, b, wfc, bfc):
    return pl.pallas_call(
        _d3_body,
        out_shape=jax.ShapeDtypeStruct((M8, 128), jnp.float32),
    )(acc, rd, hp, wl, wr, b, wfc, bfc)


def _kron8(w):
    return jnp.kron(jnp.eye(8, dtype=jnp.float32), w)


def _chan_selector(c):
    e = jnp.zeros((H, H), jnp.float32).at[c, :].set(1.0)
    return jnp.kron(jnp.eye(8, dtype=jnp.float32), e)


def _tile8(v):
    return jnp.tile(v.reshape(1, H), (1, 8)).reshape(1, 128)


def kernel(x1, x2, edge_index,
           Wl1_w2s, bl1_w2s, Wr1_w2s, Wl1_s2w, bl1_s2w, Wr1_s2w,
           Wl2_w2s, bl2_w2s, Wr2_w2s, Wl2_s2w, bl2_s2w, Wr2_s2w,
           Wl3_w2s, bl3_w2s, Wr3_w2s, Wl3_s2w, bl3_s2w, Wr3_s2w,
           Wfc, bfc):
    sweep2, sweep16_two, sweep16_one = _get_sweeps()

    src = edge_index[0].astype(jnp.int32)
    dst = edge_index[1].astype(jnp.int32)
    src2d = jnp.concatenate(
        [src, jnp.full((EP - E,), N1, jnp.int32)]).reshape(EP // CH, CH)
    dst2d = jnp.concatenate(
        [dst, jnp.full((EP - E,), N2, jnp.int32)]).reshape(EP // CH, CH)

    zeros16 = jnp.zeros((NP, H), jnp.float32)

    ones1 = jnp.ones((N1, 1), jnp.float32)
    t1 = jnp.concatenate([jnp.concatenate(
        [x1, ones1, jnp.zeros((N1, H - 2), jnp.float32)], axis=1),
        jnp.zeros((NP - N1, H), jnp.float32)], axis=0)
    t2 = jnp.concatenate([jnp.concatenate(
        [x2, ones1, jnp.zeros((N2, H - 2), jnp.float32)], axis=1),
        jnp.zeros((NP - N2, H), jnp.float32)], axis=0)
    xb1 = jnp.broadcast_to(
        jnp.concatenate([x1, jnp.zeros((NP - N1, 1), jnp.float32)], axis=0),
        (NP, H)).reshape(M8, 128)
    xb2 = jnp.broadcast_to(
        jnp.concatenate([x2, jnp.zeros((NP - N2, 1), jnp.float32)], axis=0),
        (NP, H)).reshape(M8, 128)

    s0 = _chan_selector(0)
    s1 = _chan_selector(1)

    accA1, accA2 = sweep2(src2d, dst2d, zeros16, t2, t1)

    h1, rd1, h2, rd2 = _dense1(
        accA1.reshape(M8, 128), accA2.reshape(M8, 128), xb1, xb2, s0, s1,
        _tile8(Wl1_w2s[:, 0]), _tile8(Wr1_w2s[:, 0]), _tile8(bl1_w2s),
        _tile8(Wl1_s2w[:, 0]), _tile8(Wr1_s2w[:, 0]), _tile8(bl1_s2w))

    accB1, accB2 = sweep16_two(src2d, dst2d, zeros16,
                               h2.reshape(NP, H), h1.reshape(NP, H))
    h1b, h2b = _dense2(
        accB1.reshape(M8, 128), accB2.reshape(M8, 128), rd1, rd2, h1, h2,
        _kron8(Wl2_w2s.T), _kron8(Wr2_w2s.T), _tile8(bl2_w2s),
        _kron8(Wl2_s2w.T), _kron8(Wr2_s2w.T), _tile8(bl2_s2w))

    (accC1,) = sweep16_one(src2d, dst2d, zeros16, h2b.reshape(NP, H))
    o = _dense3(accC1.reshape(NCORES, M8, 128), rd1, h1b,
                _kron8(Wl3_w2s.T), _kron8(Wr3_w2s.T), _tile8(bl3_w2s),
                _kron8(jnp.concatenate(
                    [Wfc, jnp.zeros((H - 1, H), jnp.float32)], axis=0).T),
                _tile8(jnp.concatenate(
                    [bfc, jnp.zeros((H - 1,), jnp.float32)])))
    return o.reshape(NP, H)[:N1, 0]


# speedup vs baseline: 65.3391x; 1.1428x over previous
import jax
import jax.numpy as jnp
from jax import lax
from jax.experimental import pallas as pl
from jax.experimental.pallas import tpu as pltpu
from jax.experimental.pallas import tpu_sc as plsc

N1 = 50000
N2 = 50000
E = 1600000
H = 16

NCORES = 2
NSUB = 16
NW = NCORES * NSUB
CH = 128
ROWS_E = E // CH
NP = 50176
ROWS_T = NP // NSUB


def _make_sweep_dir_per_core(w, interpret=False):
    kc = 11
    nsup = 71
    npair = 35
    rows_per_tile = ROWS_E // NSUB
    nextra = ROWS_E - rows_per_tile * NSUB
    mesh = plsc.VectorSubcoreMesh(core_axis_name="c", subcore_axis_name="s",
                                  num_cores=NCORES, num_subcores=NSUB)
    out_type = (jax.ShapeDtypeStruct((NP, w), jnp.float32),
                jax.ShapeDtypeStruct((NP, w), jnp.float32))
    scratch = [
        pltpu.VMEM((2, kc, CH), jnp.int32),
        pltpu.VMEM((2, kc, CH), jnp.int32),
        pltpu.VMEM((2, kc, CH, w), jnp.float32),
        pltpu.VMEM_SHARED((NP, w), jnp.float32),
        pltpu.SemaphoreType.DMA,
        pltpu.SemaphoreType.DMA,
        pltpu.SemaphoreType.DMA,
        pltpu.SemaphoreType.DMA,
    ]

    def body(eiH, zH, tabdH, tabsH, out1H, out2H,
             sidx, didx, g, acc, gs0, gs1, ss0, ss1):
        cid = lax.axis_index("c")
        sid = lax.axis_index("s")
        r0 = sid * ROWS_T
        pltpu.sync_copy(zH.at[pl.ds(r0, ROWS_T), :], acc.at[pl.ds(r0, ROWS_T), :])
        plsc.subcore_barrier()
        row0 = sid * rows_per_tile
        gsem = (gs0, gs1)
        ssem = (ss0, ss1)

        def pipe(gdim, scdim):
            gidx = (sidx, didx)[gdim]
            scidx = (sidx, didx)[scdim]
            tabH = (tabsH, tabdH)[gdim]

            def fire(b, rbase, n):
                pltpu.sync_copy(eiH.at[0, pl.ds(rbase, n), :], sidx.at[b, pl.ds(0, n)])
                pltpu.sync_copy(eiH.at[1, pl.ds(rbase, n), :], didx.at[b, pl.ds(0, n)])
                for k in range(n):
                    pltpu.async_copy(tabH.at[gidx.at[b, k]], g.at[b, k], gsem[b])

            def drain_gather(b, n):
                for k in range(n):
                    pltpu.make_async_copy(tabH.at[gidx.at[b, k]], g.at[b, k],
                                          gsem[b]).wait()

            def fire_scatter(b, n):
                for k in range(n):
                    pltpu.async_copy(g.at[b, k], acc.at[scidx.at[b, k]],
                                     ssem[b], add=True)

            def drain_scatter(b, n):
                for k in range(n):
                    pltpu.make_async_copy(g.at[b, k], acc.at[scidx.at[b, k]],
                                          ssem[b]).wait()

            def pair(i, carry):
                rbase = row0 + i * 2 * kc

                @pl.when(i > 0)
                def _():
                    drain_scatter(0, kc)
                fire(0, rbase, kc)

                @pl.when(i > 0)
                def _():
                    drain_scatter(1, kc)
                fire(1, rbase + kc, kc)

                drain_gather(0, kc)
                fire_scatter(0, kc)
                drain_gather(1, kc)
                fire_scatter(1, kc)
                return carry

            lax.fori_loop(0, npair, pair, 0)
            drain_scatter(0, kc)
            drain_scatter(1, kc)
            fire(0, row0 + npair * 2 * kc, kc)
            drain_gather(0, kc)
            fire_scatter(0, kc)
            drain_scatter(0, kc)

            @pl.when(sid < nextra)
            def _():
                extra = rows_per_tile * NSUB + sid
                fire(1, extra, 1)
                drain_gather(1, 1)
                fire_scatter(1, 1)
                drain_scatter(1, 1)

        @pl.when(cid == 0)
        def _():
            pipe(1, 0)

        @pl.when(cid == 1)
        def _():
            pipe(0, 1)

        plsc.subcore_barrier()

        @pl.when(cid == 0)
        def _():
            pltpu.sync_copy(acc.at[pl.ds(r0, ROWS_T), :],
                            out1H.at[pl.ds(r0, ROWS_T), :])

        @pl.when(cid == 1)
        def _():
            pltpu.sync_copy(acc.at[pl.ds(r0, ROWS_T), :],
                            out2H.at[pl.ds(r0, ROWS_T), :])

    return pl.kernel(body, out_type=out_type, mesh=mesh,
                     scratch_types=scratch, interpret=interpret,
                     compiler_params=pltpu.CompilerParams(
                         use_tc_tiling_on_sc=False))


def _make_sweep_one_dir(w, interpret=False):
    kc = 13
    npair = 15
    rows_per_w = ROWS_E // NW
    nextra = ROWS_E - rows_per_w * NW
    mesh = plsc.VectorSubcoreMesh(core_axis_name="c", subcore_axis_name="s",
                                  num_cores=NCORES, num_subcores=NSUB)
    out_type = (jax.ShapeDtypeStruct((NCORES, NP, w), jnp.float32),)
    scratch = [
        pltpu.VMEM((2, kc, CH), jnp.int32),
        pltpu.VMEM((2, kc, CH), jnp.int32),
        pltpu.VMEM((2, kc, CH, w), jnp.float32),
        pltpu.VMEM_SHARED((NP, w), jnp.float32),
        pltpu.SemaphoreType.DMA,
        pltpu.SemaphoreType.DMA,
        pltpu.SemaphoreType.DMA,
        pltpu.SemaphoreType.DMA,
    ]

    def body(eiH, zH, tabdH, out1H,
             sidx, didx, g, acc, gs0, gs1, ss0, ss1):
        cid = lax.axis_index("c")
        sid = lax.axis_index("s")
        wid = sid * NCORES + cid
        r0 = sid * ROWS_T
        pltpu.sync_copy(zH.at[pl.ds(r0, ROWS_T), :], acc.at[pl.ds(r0, ROWS_T), :])
        plsc.subcore_barrier()
        row0 = wid * rows_per_w
        gsem = (gs0, gs1)
        ssem = (ss0, ss1)

        def fire(b, rbase, n):
            pltpu.sync_copy(eiH.at[0, pl.ds(rbase, n), :], sidx.at[b, pl.ds(0, n)])
            pltpu.sync_copy(eiH.at[1, pl.ds(rbase, n), :], didx.at[b, pl.ds(0, n)])
            for k in range(n):
                pltpu.async_copy(tabdH.at[didx.at[b, k]], g.at[b, k], gsem[b])

        def drain_gather(b, n):
            for k in range(n):
                pltpu.make_async_copy(tabdH.at[didx.at[b, k]], g.at[b, k],
                                      gsem[b]).wait()

        def fire_scatter(b, n):
            for k in range(n):
                pltpu.async_copy(g.at[b, k], acc.at[sidx.at[b, k]],
                                 ssem[b], add=True)

        def drain_scatter(b, n):
            for k in range(n):
                pltpu.make_async_copy(g.at[b, k], acc.at[sidx.at[b, k]],
                                      ssem[b]).wait()

        def pair(i, carry):
            rbase = row0 + i * 2 * kc

            @pl.when(i > 0)
            def _():
                drain_scatter(0, kc)
            fire(0, rbase, kc)

            @pl.when(i > 0)
            def _():
                drain_scatter(1, kc)
            fire(1, rbase + kc, kc)

            drain_gather(0, kc)
            fire_scatter(0, kc)
            drain_gather(1, kc)
            fire_scatter(1, kc)
            return carry

        lax.fori_loop(0, npair, pair, 0)
        drain_scatter(0, kc)
        drain_scatter(1, kc)

        @pl.when(wid < nextra)
        def _():
            extra = rows_per_w * NW + wid
            fire(0, extra, 1)
            drain_gather(0, 1)
            fire_scatter(0, 1)
            drain_scatter(0, 1)

        plsc.subcore_barrier()
        pltpu.sync_copy(acc.at[pl.ds(r0, ROWS_T), :],
                        out1H.at[cid, pl.ds(r0, ROWS_T), :])

    return pl.kernel(body, out_type=out_type, mesh=mesh,
                     scratch_types=scratch, interpret=interpret,
                     compiler_params=pltpu.CompilerParams(
                         use_tc_tiling_on_sc=False))


_sweeps = None


def _get_sweeps():
    global _sweeps
    if _sweeps is None:
        _sweeps = (_make_sweep_dir_per_core(8),
                   _make_sweep_dir_per_core(H),
                   _make_sweep_one_dir(H))
    return _sweeps


M8 = NP // 8


def _d1_body(sb1_ref, db1_ref, sb2_ref, db2_ref, xb1_ref, xb2_ref,
             wl1_ref, wr1_ref, b1_ref, wl2_ref, wr2_ref, b2_ref,
             h1_ref, rd1_ref, h2_ref, rd2_ref):
    def side(sb_ref, db_ref, xb_ref, wl_ref, wr_ref, b_ref, h_ref, rd_ref):
        rd = 1.0 / jnp.maximum(db_ref[...], 1.0)
        h = (sb_ref[...] * rd) * wl_ref[...] + xb_ref[...] * wr_ref[...] + b_ref[...]
        h_ref[...] = jnp.maximum(h, 0.0)
        rd_ref[...] = rd

    side(sb1_ref, db1_ref, xb1_ref, wl1_ref, wr1_ref, b1_ref, h1_ref, rd1_ref)
    side(sb2_ref, db2_ref, xb2_ref, wl2_ref, wr2_ref, b2_ref, h2_ref, rd2_ref)


def _dense1(sb1, db1, sb2, db2, xb1, xb2,
            wl1, wr1, b1, wl2, wr2, b2):
    return pl.pallas_call(
        _d1_body,
        out_shape=[
            jax.ShapeDtypeStruct((M8, 128), jnp.float32),
            jax.ShapeDtypeStruct((M8, 128), jnp.float32),
            jax.ShapeDtypeStruct((M8, 128), jnp.float32),
            jax.ShapeDtypeStruct((M8, 128), jnp.float32),
        ],
    )(sb1, db1, sb2, db2, xb1, xb2, wl1, wr1, b1, wl2, wr2, b2)


def _d2_body(a1_ref, a2_ref, rd1_ref, rd2_ref, h1_ref, h2_ref,
             wl1_ref, wr1_ref, b1_ref, wl2_ref, wr2_ref, b2_ref,
             o1_ref, o2_ref):
    m1 = a1_ref[...] * rd1_ref[...]
    o1 = (jnp.dot(m1, wl1_ref[...], preferred_element_type=jnp.float32)
          + jnp.dot(h1_ref[...], wr1_ref[...], preferred_element_type=jnp.float32)
          + b1_ref[...])
    o1_ref[...] = jnp.maximum(o1, 0.0)
    m2 = a2_ref[...] * rd2_ref[...]
    o2 = (jnp.dot(m2, wl2_ref[...], preferred_element_type=jnp.float32)
          + jnp.dot(h2_ref[...], wr2_ref[...], preferred_element_type=jnp.float32)
          + b2_ref[...])
    o2_ref[...] = jnp.maximum(o2, 0.0)


def _dense2(a1, a2, rd1, rd2, h1, h2, wl1, wr1, b1, wl2, wr2, b2):
    return pl.pallas_call(
        _d2_body,
        out_shape=[
            jax.ShapeDtypeStruct((M8, 128), jnp.float32),
            jax.ShapeDtypeStruct((M8, 128), jnp.float32),
        ],
    )(a1, a2, rd1, rd2, h1, h2, wl1, wr1, b1, wl2, wr2, b2)


def _d3_body(acc_ref, rd_ref, hp_ref, wl_ref, wr_ref, b_ref,
             wfc_ref, bfc_ref, o_ref):
    m = (acc_ref[0] + acc_ref[1]) * rd_ref[...]
    g = (jnp.dot(m, wl_ref[...], preferred_element_type=jnp.float32)
         + jnp.dot(hp_ref[...], wr_ref[...], preferred_element_type=jnp.float32)
         + b_ref[...])
    g = jnp.maximum(g, 0.0)
    z = jnp.dot(g, wfc_ref[...], preferred_element_type=jnp.float32) + bfc_ref[...]
    o_ref[...] = 1.0 / (1.0 + jnp.exp(-z))


def _dense3(acc, rd, hp, wl, wr, b, wfc, bfc):
    return pl.pallas_call(
        _d3_body,
        out_shape=jax.ShapeDtypeStruct((M8, 8), jnp.float32),
    )(acc, rd, hp, wl, wr, b, wfc, bfc)


def _kron8(w):
    return jnp.kron(jnp.eye(8, dtype=jnp.float32), w)


def _tile8(v):
    return jnp.tile(v.reshape(1, H), (1, 8)).reshape(1, 128)


def kernel(x1, x2, edge_index,
           Wl1_w2s, bl1_w2s, Wr1_w2s, Wl1_s2w, bl1_s2w, Wr1_s2w,
           Wl2_w2s, bl2_w2s, Wr2_w2s, Wl2_s2w, bl2_s2w, Wr2_s2w,
           Wl3_w2s, bl3_w2s, Wr3_w2s, Wl3_s2w, bl3_s2w, Wr3_s2w,
           Wfc, bfc):
    sweep2, sweep16_two, sweep16_one = _get_sweeps()

    ei3d = edge_index.astype(jnp.int32).reshape(2, ROWS_E, CH)

    zeros16 = jnp.zeros((NP, H), jnp.float32)
    zeros8 = jnp.zeros((NP, 8), jnp.float32)

    ones1 = jnp.ones((N1, 1), jnp.float32)
    t1 = jnp.concatenate([jnp.concatenate(
        [x1, ones1, jnp.zeros((N1, 6), jnp.float32)], axis=1),
        jnp.zeros((NP - N1, 8), jnp.float32)], axis=0)
    t2 = jnp.concatenate([jnp.concatenate(
        [x2, ones1, jnp.zeros((N2, 6), jnp.float32)], axis=1),
        jnp.zeros((NP - N2, 8), jnp.float32)], axis=0)
    xb1 = jnp.broadcast_to(
        jnp.concatenate([x1, jnp.zeros((NP - N1, 1), jnp.float32)], axis=0),
        (NP, H)).reshape(M8, 128)
    xb2 = jnp.broadcast_to(
        jnp.concatenate([x2, jnp.zeros((NP - N2, 1), jnp.float32)], axis=0),
        (NP, H)).reshape(M8, 128)

    accA1, accA2 = sweep2(ei3d, zeros8, t2, t1)

    def _scal(a, c):
        return jnp.broadcast_to(a[:, c:c + 1], (NP, H)).reshape(M8, 128)

    h1, rd1, h2, rd2 = _dense1(
        _scal(accA1, 0), _scal(accA1, 1), _scal(accA2, 0), _scal(accA2, 1),
        xb1, xb2,
        _tile8(Wl1_w2s[:, 0]), _tile8(Wr1_w2s[:, 0]), _tile8(bl1_w2s),
        _tile8(Wl1_s2w[:, 0]), _tile8(Wr1_s2w[:, 0]), _tile8(bl1_s2w))

    accB1, accB2 = sweep16_two(ei3d, zeros16,
                               h2.reshape(NP, H), h1.reshape(NP, H))
    h1b, h2b = _dense2(
        accB1.reshape(M8, 128), accB2.reshape(M8, 128), rd1, rd2, h1, h2,
        _kron8(Wl2_w2s.T), _kron8(Wr2_w2s.T), _tile8(bl2_w2s),
        _kron8(Wl2_s2w.T), _kron8(Wr2_s2w.T), _tile8(bl2_s2w))

    (accC1,) = sweep16_one(ei3d, zeros16, h2b.reshape(NP, H))
    wsel = jnp.kron(jnp.eye(8, dtype=jnp.float32), Wfc.T)
    o = _dense3(accC1.reshape(NCORES, M8, 128), rd1, h1b,
                _kron8(Wl3_w2s.T), _kron8(Wr3_w2s.T), _tile8(bl3_w2s),
                wsel, jnp.broadcast_to(bfc.reshape(1, 1), (1, 8)))
    return o.reshape(NP)[:N1]
